# baseline (speedup 1.0000x reference)
import functools

import jax
import jax.numpy as jnp
from jax import lax
from jax.experimental import pallas as pl
from jax.experimental.pallas import tpu as pltpu
from jax.experimental.pallas import tpu_sc as plsc

D = 32
NT = 200
NB = 16384
TT = NT // 8
TB = NB // 128
NUM_CORES = 2
NUM_SUBCORES = 16
NUM_WORKERS = NUM_CORES * NUM_SUBCORES
QB = TB // NUM_WORKERS
NS = 8


@jax.jit
def _lookup(tok_p, weight):
    mesh = plsc.VectorSubcoreMesh(core_axis_name="c", subcore_axis_name="s")

    @functools.partial(
        pl.kernel,
        mesh=mesh,
        out_type=jax.ShapeDtypeStruct((NT, D // 8, TB, 8, 128), jnp.float32),
        scratch_types=[
            pltpu.VMEM((TT, 8, 128), jnp.int32),
            pltpu.VMEM((NS, 128, D), jnp.float32),
            pltpu.VMEM((2, D, 128), jnp.float32),
            pltpu.SemaphoreType.DMA((NS,)),
            pltpu.SemaphoreType.DMA((2,)),
        ],
        compiler_params=pltpu.CompilerParams(
            use_tc_tiling_on_sc=False, needs_layout_passes=False
        ),
    )
    def k(tok_hbm, w_hbm, out_hbm, tokb, rows, outt, gsem, wsem):
        wid = lax.axis_index("s") * NUM_CORES + lax.axis_index("c")
        iota = lax.iota(jnp.int32, 16)
        row_lo = iota
        row_hi = iota + 16

        def fire(t, s):
            tt = t // 8
            r = t % 8
            pltpu.async_copy(
                w_hbm.at[tokb.at[tt, r]], rows.at[s], gsem.at[s]
            )

        def wait_g(s):
            pltpu.make_async_copy(
                w_hbm.at[pl.ds(0, 128)], rows.at[s], gsem.at[s]
            ).wait()

        def transpose(s, w):
            rs = rows.at[s]
            ow = outt.at[w]
            rowvs = [iota + c0 for c0 in range(0, 128, 16)]
            for d0 in range(0, D, 16):
                for j in range(16):
                    colv = ((iota + j) & 15) + d0
                    for rowv in rowvs:
                        v = plsc.load_gather(rs, [rowv, colv])
                        plsc.store_scatter(ow, [colv, rowv], v)

        def fire_w(t, tb, w):
            for tr in range(D // 8):
                pltpu.async_copy(
                    outt.at[w, pl.ds(tr * 8, 8)], out_hbm.at[t, tr, tb],
                    wsem.at[w],
                )

        def wait_w(w):
            for tr in range(D // 8):
                pltpu.make_async_copy(
                    outt.at[w, pl.ds(tr * 8, 8)], out_hbm.at[0, tr, 0],
                    wsem.at[w],
                ).wait()

        @pl.loop(0, QB)
        def _(q):
            tb = wid * QB + q
            pltpu.sync_copy(tok_hbm.at[:, tb], tokb)
            for s0 in range(NS):
                fire(s0, s0)

            @pl.loop(0, NT)
            def _(t):
                s = t % NS
                w = t % 2

                @pl.when(t >= 2)
                def _():
                    wait_w(w)

                wait_g(s)
                transpose(s, w)

                @pl.when(t < NT - NS)
                def _():
                    fire(t + NS, s)

                fire_w(t, tb, w)

            for w0 in range(2):
                wait_w(w0)

    return k(tok_p, weight)


def kernel(token_ids, weight):
    tok = token_ids.astype(jnp.int32)
    tok_p = jnp.transpose(tok).reshape(TT, 8, TB, 128).transpose(0, 2, 1, 3)
    out_p = _lookup(tok_p, weight)
    return (out_p.transpose(0, 1, 3, 2, 4)
            .reshape(NT, D, NB)
            .transpose(2, 0, 1))


# speedup vs baseline: 1.6654x; 1.6654x over previous
import functools

import jax
import jax.numpy as jnp
from jax import lax
from jax.experimental import pallas as pl
from jax.experimental.pallas import tpu as pltpu
from jax.experimental.pallas import tpu_sc as plsc

D = 32
NT = 200
NB = 16384
TT = NT // 8
TB = NB // 128
NUM_CORES = 2
NUM_SUBCORES = 16
NUM_WORKERS = NUM_CORES * NUM_SUBCORES
QB = TB // NUM_WORKERS
NS = 8


@jax.jit
def _lookup(tok_p, weight):
    mesh = plsc.VectorSubcoreMesh(core_axis_name="c", subcore_axis_name="s")

    @functools.partial(
        pl.kernel,
        mesh=mesh,
        out_type=jax.ShapeDtypeStruct((NT, D // 8, TB, 8, 128), jnp.float32),
        scratch_types=[
            pltpu.VMEM((TT, 8, 128), jnp.int32),
            pltpu.VMEM((NS * 128, D), jnp.float32),
            pltpu.VMEM((2 * D, 128), jnp.float32),
            pltpu.SemaphoreType.DMA((NS,)),
            pltpu.SemaphoreType.DMA((2,)),
        ],
        compiler_params=pltpu.CompilerParams(
            use_tc_tiling_on_sc=False, needs_layout_passes=False
        ),
    )
    def k(tok_hbm, w_hbm, out_hbm, tokb, rows, outt, gsem, wsem):
        wid = lax.axis_index("s") * NUM_CORES + lax.axis_index("c")
        iota = lax.iota(jnp.int32, 16)
        row_lo = iota
        row_hi = iota + 16

        def fire(t, s):
            tt = t // 8
            r = t % 8
            pltpu.async_copy(
                w_hbm.at[tokb.at[tt, r]], rows.at[pl.ds(s * 128, 128)],
                gsem.at[s],
            )

        def wait_g(s):
            pltpu.make_async_copy(
                w_hbm.at[pl.ds(0, 128)], rows.at[pl.ds(0, 128)], gsem.at[s]
            ).wait()

        def transpose(s, w):
            rb = s * 128
            wb = w * D
            for c0 in range(0, 128, 16):
                rowv = iota + (rb + c0)
                for d0 in range(0, D, 16):
                    for j in range(16):
                        rot = (iota + j) & 15
                        v = plsc.load_gather(rows, [rowv, rot + d0])
                        plsc.store_scatter(
                            outt, [rot + (wb + d0), iota + c0], v
                        )

        def fire_w(t, tb, w):
            for tr in range(D // 8):
                pltpu.async_copy(
                    outt.at[pl.ds(w * D + tr * 8, 8)], out_hbm.at[t, tr, tb],
                    wsem.at[w],
                )

        def wait_w(w):
            for tr in range(D // 8):
                pltpu.make_async_copy(
                    outt.at[pl.ds(tr * 8, 8)], out_hbm.at[0, tr, 0],
                    wsem.at[w],
                ).wait()

        @pl.loop(0, QB)
        def _(q):
            tb = wid * QB + q
            pltpu.sync_copy(tok_hbm.at[:, tb], tokb)
            for s0 in range(NS):
                fire(s0, s0)

            @pl.loop(0, NT)
            def _(t):
                s = t % NS
                w = t % 2

                @pl.when(t >= 2)
                def _():
                    wait_w(w)

                wait_g(s)
                transpose(s, w)

                @pl.when(t < NT - NS)
                def _():
                    fire(t + NS, s)

                fire_w(t, tb, w)

            for w0 in range(2):
                wait_w(w0)

    return k(tok_p, weight)


def kernel(token_ids, weight):
    tok = token_ids.astype(jnp.int32)
    tok_p = jnp.transpose(tok).reshape(TT, 8, TB, 128).transpose(0, 2, 1, 3)
    out_p = _lookup(tok_p, weight)
    return (out_p.transpose(0, 1, 3, 2, 4)
            .reshape(NT, D, NB)
            .transpose(2, 0, 1))


# speedup vs baseline: 1.7556x; 1.0542x over previous
import functools

import jax
import jax.numpy as jnp
from jax import lax
from jax.experimental import pallas as pl
from jax.experimental.pallas import tpu as pltpu
from jax.experimental.pallas import tpu_sc as plsc

D = 32
NT = 200
NB = 16384
TT = NT // 8
TB = NB // 128
NUM_CORES = 2
NUM_SUBCORES = 16
NUM_WORKERS = NUM_CORES * NUM_SUBCORES
QB = TB // NUM_WORKERS
NS = 8


@jax.jit
def _lookup(tok_p, weight):
    mesh = plsc.VectorSubcoreMesh(core_axis_name="c", subcore_axis_name="s")

    @functools.partial(
        pl.kernel,
        mesh=mesh,
        out_type=jax.ShapeDtypeStruct((NT, D // 8, TB, 8, 128), jnp.float32),
        scratch_types=[
            pltpu.VMEM((TT, 8, 128), jnp.int32),
            pltpu.VMEM((NS * 128, D), jnp.float32),
            pltpu.VMEM((2 * D, 128), jnp.float32),
            pltpu.SemaphoreType.DMA((NS,)),
            pltpu.SemaphoreType.DMA((2,)),
        ],
        compiler_params=pltpu.CompilerParams(
            use_tc_tiling_on_sc=False, needs_layout_passes=False
        ),
    )
    def k(tok_hbm, w_hbm, out_hbm, tokb, rows, outt, gsem, wsem):
        wid = lax.axis_index("s") * NUM_CORES + lax.axis_index("c")
        iota = lax.iota(jnp.int32, 16)
        row_lo = iota
        row_hi = iota + 16

        def fire(t, s):
            tt = t // 8
            r = t % 8
            pltpu.async_copy(
                w_hbm.at[tokb.at[tt, r]], rows.at[pl.ds(s * 128, 128)],
                gsem.at[s],
            )

        def wait_g(s):
            pltpu.make_async_copy(
                w_hbm.at[pl.ds(0, 128)], rows.at[pl.ds(0, 128)], gsem.at[s]
            ).wait()

        def transpose(s, w):
            rb = s * 128
            wb = w * D
            rowvs = [iota + c0 for c0 in range(0, 128, 16)]
            growvs = [rv + rb for rv in rowvs]
            for d0 in range(0, D, 16):
                for j in range(16):
                    rot = (iota + j) & 15
                    gcol = rot + d0
                    scol = rot + (wb + d0)
                    for ci in range(8):
                        v = plsc.load_gather(rows, [growvs[ci], gcol])
                        plsc.store_scatter(outt, [scol, rowvs[ci]], v)

        def fire_w(t, tb, w):
            for tr in range(D // 8):
                pltpu.async_copy(
                    outt.at[pl.ds(w * D + tr * 8, 8)], out_hbm.at[t, tr, tb],
                    wsem.at[w],
                )

        def wait_w(w):
            for tr in range(D // 8):
                pltpu.make_async_copy(
                    outt.at[pl.ds(tr * 8, 8)], out_hbm.at[0, tr, 0],
                    wsem.at[w],
                ).wait()

        @pl.loop(0, QB)
        def _(q):
            tb = wid * QB + q
            pltpu.sync_copy(tok_hbm.at[:, tb], tokb)
            for s0 in range(NS):
                fire(s0, s0)

            @pl.loop(0, NT)
            def _(t):
                s = t % NS
                w = t % 2

                @pl.when(t >= 2)
                def _():
                    wait_w(w)

                wait_g(s)
                transpose(s, w)

                @pl.when(t < NT - NS)
                def _():
                    fire(t + NS, s)

                fire_w(t, tb, w)

            for w0 in range(2):
                wait_w(w0)

    return k(tok_p, weight)


def kernel(token_ids, weight):
    tok = token_ids.astype(jnp.int32)
    tok_p = jnp.transpose(tok).reshape(TT, 8, TB, 128).transpose(0, 2, 1, 3)
    out_p = _lookup(tok_p, weight)
    return (out_p.transpose(0, 1, 3, 2, 4)
            .reshape(NT, D, NB)
            .transpose(2, 0, 1))


# speedup vs baseline: 1.7556x; 1.0000x over previous
import functools

import jax
import jax.numpy as jnp
from jax import lax
from jax.experimental import pallas as pl
from jax.experimental.pallas import tpu as pltpu
from jax.experimental.pallas import tpu_sc as plsc

D = 32
NT = 200
NB = 16384
TT = NT // 8
TB = NB // 128
NUM_CORES = 2
NUM_SUBCORES = 16
NUM_WORKERS = NUM_CORES * NUM_SUBCORES
QB = TB // NUM_WORKERS
NS = 8


@jax.jit
def _lookup(tok_p, weight):
    mesh = plsc.VectorSubcoreMesh(core_axis_name="c", subcore_axis_name="s")

    @functools.partial(
        pl.kernel,
        mesh=mesh,
        out_type=jax.ShapeDtypeStruct((NT, D // 8, TB, 8, 128), jnp.float32),
        scratch_types=[
            pltpu.VMEM((TT, 8, 128), jnp.int32),
            pltpu.VMEM((NS * 128, D), jnp.float32),
            pltpu.VMEM((2 * D, 128), jnp.float32),
            pltpu.SemaphoreType.DMA((NS,)),
            pltpu.SemaphoreType.DMA((2,)),
        ],
        compiler_params=pltpu.CompilerParams(
            use_tc_tiling_on_sc=False, needs_layout_passes=False
        ),
    )
    def k(tok_hbm, w_hbm, out_hbm, tokb, rows, outt, gsem, wsem):
        wid = lax.axis_index("s") * NUM_CORES + lax.axis_index("c")
        iota = lax.iota(jnp.int32, 16)

        def fire(t, s):
            tt = t // 8
            r = t % 8
            pltpu.async_copy(
                w_hbm.at[tokb.at[tt, r]], rows.at[pl.ds(s * 128, 128)],
                gsem.at[s],
            )

        def wait_g(s):
            pltpu.make_async_copy(
                w_hbm.at[pl.ds(0, 128)], rows.at[pl.ds(0, 128)], gsem.at[s]
            ).wait()

        def transpose(s, w):
            rb = s * 128
            wb = w * D
            rowvs = [iota + c0 for c0 in range(0, 128, 16)]
            growvs = [rv + rb for rv in rowvs]
            for d0 in range(0, D, 16):
                for j in range(16):
                    rot = (iota + j) & 15
                    gcol = rot + d0
                    scol = rot + (wb + d0)
                    for ci in range(8):
                        v = plsc.load_gather(rows, [growvs[ci], gcol])
                        plsc.store_scatter(outt, [scol, rowvs[ci]], v)

        def fire_w(t, tb, w):
            for tr in range(D // 8):
                pltpu.async_copy(
                    outt.at[pl.ds(w * D + tr * 8, 8)], out_hbm.at[t, tr, tb],
                    wsem.at[w],
                )

        def wait_w(w):
            for tr in range(D // 8):
                pltpu.make_async_copy(
                    outt.at[pl.ds(tr * 8, 8)], out_hbm.at[0, tr, 0],
                    wsem.at[w],
                ).wait()

        @pl.loop(0, QB)
        def _(q):
            tb = wid * QB + q
            pltpu.sync_copy(tok_hbm.at[:, tb], tokb)
            for s0 in range(NS):
                fire(s0, s0)

            @pl.loop(0, NT)
            def _(t):
                s = t % NS
                w = t % 2

                @pl.when(t >= 2)
                def _():
                    wait_w(w)

                wait_g(s)
                transpose(s, w)

                @pl.when(t < NT - NS)
                def _():
                    fire(t + NS, s)

                fire_w(t, tb, w)

            for w0 in range(2):
                wait_w(w0)

    return k(tok_p, weight)


def kernel(token_ids, weight):
    assert token_ids.shape == (NB, NT) and weight.shape[1] == D
    tok = token_ids.astype(jnp.int32)
    tok_p = jnp.transpose(tok).reshape(TT, 8, TB, 128).transpose(0, 2, 1, 3)
    out_p = _lookup(tok_p, weight)
    return (out_p.transpose(0, 1, 3, 2, 4)
            .reshape(NT, D, NB)
            .transpose(2, 0, 1))
